# baseline (speedup 1.0000x reference)
import jax
import jax.numpy as jnp
from jax import lax
from jax.experimental import pallas as pl
from jax.experimental.pallas import tpu as pltpu

N, F, K, C = 18432, 768, 64, 8192
BETA = 0.25

M = 256
CB = 2048
CA = 2048


def _rowsum_sq(x):
    v = x * x
    acc = v[:, 0:8]
    for c in range(1, 8):
        acc = acc + v[:, 8 * c:8 * (c + 1)]
    a4 = acc[:, 0:4] + acc[:, 4:8]
    a2 = a4[:, 0:2] + a4[:, 2:4]
    return a2[:, 0:1] + a2[:, 1:2]


def _cbn_body(cb_ref, cbn_ref, cbsq_ref):
    cb = cb_ref[...]
    norm = jnp.sqrt(_rowsum_sq(cb))
    cbn = cb / (norm + 1e-12)
    cbn_ref[...] = cbn
    cbsq_ref[...] = _rowsum_sq(cbn)


def _normalise_codebook(codebook):
    return pl.pallas_call(
        _cbn_body,
        grid=(C // CA,),
        in_specs=[pl.BlockSpec((CA, K), lambda i: (i, 0))],
        out_specs=[
            pl.BlockSpec((CA, K), lambda i: (i, 0)),
            pl.BlockSpec((CA, 1), lambda i: (i, 0)),
        ],
        out_shape=[
            jax.ShapeDtypeStruct((C, K), jnp.float32),
            jax.ShapeDtypeStruct((C, 1), jnp.float32),
        ],
    )(codebook)


def _zq_body(z_ref, Wi_ref, bi_ref, zq_ref, zqsq_ref):
    acc = jnp.dot(z_ref[:, 0:256], Wi_ref[0:256, :],
                  preferred_element_type=jnp.float32)
    acc = acc + jnp.dot(z_ref[:, 256:512], Wi_ref[256:512, :],
                        preferred_element_type=jnp.float32)
    acc = acc + jnp.dot(z_ref[:, 512:768], Wi_ref[512:768, :],
                        preferred_element_type=jnp.float32)
    x = acc + bi_ref[...]
    norm = jnp.sqrt(_rowsum_sq(x))
    zq = x / (norm + 1e-12)
    zq_ref[...] = zq
    zqsq_ref[...] = _rowsum_sq(zq)


def _project(z, W_in, b_in2):
    return pl.pallas_call(
        _zq_body,
        grid=(N // M,),
        in_specs=[
            pl.BlockSpec((M, F), lambda i: (i, 0)),
            pl.BlockSpec((F, K), lambda i: (0, 0)),
            pl.BlockSpec((1, K), lambda i: (0, 0)),
        ],
        out_specs=[
            pl.BlockSpec((M, K), lambda i: (i, 0)),
            pl.BlockSpec((M, 1), lambda i: (i, 0)),
        ],
        out_shape=[
            jax.ShapeDtypeStruct((N, K), jnp.float32),
            jax.ShapeDtypeStruct((N, 1), jnp.float32),
        ],
    )(z, W_in, b_in2)


def _argmin_body(zq_ref, zqsq_ref, cbnT_ref, cbsq_ref,
                 idx_ref, rmin_s, ridx_s):
    j = pl.program_id(1)
    nj = pl.num_programs(1)

    @pl.when(j == 0)
    def _init():
        rmin_s[...] = jnp.full((M, 1), jnp.inf, jnp.float32)
        ridx_s[...] = jnp.zeros((M, 1), jnp.int32)

    sim = jnp.dot(zq_ref[...], cbnT_ref[...],
                  preferred_element_type=jnp.float32)
    dist = zqsq_ref[...] - 2.0 * sim + cbsq_ref[...]
    blk_min = jnp.min(dist, axis=-1, keepdims=True)
    cols = lax.broadcasted_iota(jnp.int32, (M, CB), 1) + j * CB
    blk_idx = jnp.min(jnp.where(dist == blk_min, cols, jnp.int32(2**30)),
                      axis=-1, keepdims=True)
    better = blk_min < rmin_s[...]
    rmin_s[...] = jnp.where(better, blk_min, rmin_s[...])
    ridx_s[...] = jnp.where(better, blk_idx, ridx_s[...])

    @pl.when(j == nj - 1)
    def _fin():
        idx_ref[...] = ridx_s[...]


def _argmin(zq, zqsq, cbnT, cbsq_row):
    return pl.pallas_call(
        _argmin_body,
        grid=(N // M, C // CB),
        in_specs=[
            pl.BlockSpec((M, K), lambda i, j: (i, 0)),
            pl.BlockSpec((M, 1), lambda i, j: (i, 0)),
            pl.BlockSpec((K, CB), lambda i, j: (0, j)),
            pl.BlockSpec((1, CB), lambda i, j: (0, j)),
        ],
        out_specs=pl.BlockSpec((M, 1), lambda i, j: (i, 0)),
        out_shape=jax.ShapeDtypeStruct((N, 1), jnp.int32),
        scratch_shapes=[
            pltpu.VMEM((M, 1), jnp.float32),
            pltpu.VMEM((M, 1), jnp.int32),
        ],
        compiler_params=pltpu.CompilerParams(
            dimension_semantics=("arbitrary", "arbitrary")),
    )(zq, zqsq, cbnT, cbsq_row)


def _tail_body(idx_ref, cbn_ref, zq_ref, Wo_ref, bo_ref,
               out_ref, loss_ref, codes_s):
    def gather_row(r, carry):
        codes_s[pl.ds(r, 1), :] = cbn_ref[pl.ds(idx_ref[r, 0], 1), :]
        return carry
    lax.fori_loop(0, M, gather_row, 0)
    codes = codes_s[...]
    zq = zq_ref[...]
    d1 = zq - codes
    loss_ref[...] = BETA * (d1 * d1) + (d1 * d1)
    cs = zq + (codes - zq)
    out_ref[...] = jnp.dot(cs, Wo_ref[...],
                           preferred_element_type=jnp.float32) + bo_ref[...]


def _tail(idx2, cbn, zq, W_out, b_out2):
    return pl.pallas_call(
        _tail_body,
        grid=(N // M,),
        in_specs=[
            pl.BlockSpec((M, 1), lambda i: (i, 0), memory_space=pltpu.SMEM),
            pl.BlockSpec((C, K), lambda i: (0, 0)),
            pl.BlockSpec((M, K), lambda i: (i, 0)),
            pl.BlockSpec((K, F), lambda i: (0, 0)),
            pl.BlockSpec((1, F), lambda i: (0, 0)),
        ],
        out_specs=[
            pl.BlockSpec((M, F), lambda i: (i, 0)),
            pl.BlockSpec((M, K), lambda i: (i, 0)),
        ],
        out_shape=[
            jax.ShapeDtypeStruct((N, F), jnp.float32),
            jax.ShapeDtypeStruct((N, K), jnp.float32),
        ],
        scratch_shapes=[pltpu.VMEM((M, K), jnp.float32)],
    )(idx2, cbn, zq, W_out, b_out2)


def kernel(z, W_in, b_in, codebook, W_out, b_out):
    cbn, cbsq = _normalise_codebook(codebook)
    zq, zqsq = _project(z, W_in, b_in.reshape(1, K))
    idx2 = _argmin(zq, zqsq, cbn.T, cbsq.T)
    out, loss = _tail(idx2, cbn, zq, W_out, b_out.reshape(1, F))
    return out, loss, idx2.reshape(N)
